# Initial kernel scaffold; baseline (speedup 1.0000x reference)
import jax
import jax.numpy as jnp
from jax.experimental import pallas as pl


def kernel(x, edge_index, W, b):
    raise NotImplementedError("write your pallas kernel here")


# speedup vs baseline: 5.5691x; 5.5691x over previous
import functools

import jax
import jax.numpy as jnp
from jax import lax
from jax.experimental import pallas as pl
from jax.experimental.pallas import tpu as pltpu
from jax.experimental.pallas import tpu_sc as plsc

N = 10000
D = 128
E = 320000

NC = 2
NS = 16
NW = NC * NS
CHUNK = 128
NCHUNK = -(-E // (NW * CHUNK))
E_PAD = NW * NCHUNK * CHUNK
N_PAD = 10240
STRIPE = N_PAD // NS
ROWS_PER_TILE_COPY = 128


def _sc_aggregate_kernel(x_hbm, src_hbm, dst_hbm, sums_hbm, counts_hbm,
                         src_v, dst_v, counts_v, gbuf0, gbuf1, sums_acc,
                         sem0, sem1):
    c = lax.axis_index("c")
    s = lax.axis_index("s")
    wid = s * NC + c

    pltpu.sync_copy(src_hbm.at[wid], src_v)
    pltpu.sync_copy(dst_hbm.at[wid], dst_v)

    zeros16 = jnp.zeros((16,), jnp.float32)

    def _zrow(i, _):
        for k in range(D // 16):
            gbuf0[i, pl.ds(k * 16, 16)] = zeros16
        return 0
    lax.fori_loop(0, ROWS_PER_TILE_COPY, _zrow, 0)

    def _zcnt(i, _):
        counts_v[pl.ds(i * 16, 16)] = zeros16
        return 0
    lax.fori_loop(0, N_PAD // 16, _zcnt, 0)

    for k in range(STRIPE // ROWS_PER_TILE_COPY):
        pltpu.sync_copy(
            gbuf0,
            sums_acc.at[pl.ds(s * STRIPE + k * ROWS_PER_TILE_COPY,
                              ROWS_PER_TILE_COPY)])

    plsc.subcore_barrier()

    ones16 = jnp.ones((16,), jnp.float32)

    def _chunk(j, _):
        pltpu.async_copy(x_hbm.at[src_v.at[j]], gbuf0, sem0).wait()
        pltpu.sync_copy(gbuf0, sums_acc.at[dst_v.at[j]], add=True)
        for k in range(CHUNK // 16):
            idx = dst_v.at[j][pl.ds(k * 16, 16)]
            plsc.addupdate_scatter(counts_v, [idx], ones16)
        return 0

    lax.fori_loop(0, NCHUNK, _chunk, 0)

    plsc.subcore_barrier()

    pltpu.sync_copy(sums_acc.at[pl.ds(s * STRIPE, STRIPE)],
                    sums_hbm.at[c, pl.ds(s * STRIPE, STRIPE)])
    pltpu.sync_copy(counts_v, counts_hbm.at[wid])


def _sc_aggregate(x, src_r, dst_r):
    mesh = plsc.VectorSubcoreMesh(core_axis_name="c", subcore_axis_name="s")
    return pl.kernel(
        _sc_aggregate_kernel,
        out_type=(
            jax.ShapeDtypeStruct((NC, N_PAD, D), jnp.float32),
            jax.ShapeDtypeStruct((NW, N_PAD), jnp.float32),
        ),
        mesh=mesh,
        scratch_types=[
            pltpu.VMEM((NCHUNK, CHUNK), jnp.int32),
            pltpu.VMEM((NCHUNK, CHUNK), jnp.int32),
            pltpu.VMEM((N_PAD,), jnp.float32),
            pltpu.VMEM((CHUNK, D), jnp.float32),
            pltpu.VMEM((CHUNK, D), jnp.float32),
            pltpu.VMEM_SHARED((N_PAD, D), jnp.float32),
            pltpu.SemaphoreType.DMA,
            pltpu.SemaphoreType.DMA,
        ],
        compiler_params=pltpu.CompilerParams(needs_layout_passes=False),
    )(x, src_r, dst_r)


def _tc_finish_kernel(sums_ref, counts_ref, x_ref, wt_ref, b_ref, out_ref):
    s = sums_ref[0] + sums_ref[1]
    cnt = jnp.sum(counts_ref[...], axis=0)
    agg = (s + x_ref[...]) / (cnt[:, None] + 1.0)
    acc = jnp.dot(agg, wt_ref[...], preferred_element_type=jnp.float32,
                  precision=jax.lax.Precision.HIGHEST)
    out_ref[...] = jnp.maximum(acc + b_ref[...], 0.0)


def _tc_finish(sums_p, counts_p, x_pad, wt, b2):
    blk = 1024
    grid = N_PAD // blk
    return pl.pallas_call(
        _tc_finish_kernel,
        grid=(grid,),
        in_specs=[
            pl.BlockSpec((NC, blk, D), lambda i: (0, i, 0)),
            pl.BlockSpec((NW, blk), lambda i: (0, i)),
            pl.BlockSpec((blk, D), lambda i: (i, 0)),
            pl.BlockSpec((D, D), lambda i: (0, 0)),
            pl.BlockSpec((1, D), lambda i: (0, 0)),
        ],
        out_specs=pl.BlockSpec((blk, D), lambda i: (i, 0)),
        out_shape=jax.ShapeDtypeStruct((N_PAD, D), jnp.float32),
    )(sums_p, counts_p, x_pad, wt, b2)


def kernel(x, edge_index, W, b):
    src = edge_index[0]
    dst = edge_index[1]
    pad = E_PAD - E
    src_p = jnp.concatenate([src, jnp.zeros((pad,), jnp.int32)])
    dst_p = jnp.concatenate([dst, jnp.full((pad,), N, jnp.int32)])
    src_r = src_p.reshape(NW, NCHUNK, CHUNK)
    dst_r = dst_p.reshape(NW, NCHUNK, CHUNK)
    sums_p, counts_p = _sc_aggregate(x, src_r, dst_r)
    x_pad = jnp.concatenate([x, jnp.zeros((N_PAD - N, D), jnp.float32)])
    out = _tc_finish(sums_p, counts_p, x_pad, W.T, b.reshape(1, D))
    return out[:N]
